# Initial kernel scaffold; baseline (speedup 1.0000x reference)
import jax
import jax.numpy as jnp
from jax.experimental import pallas as pl


def kernel(x, edge_index, W1, b1, W2, b2, W3, b3, Wc, bc):
    raise NotImplementedError("write your pallas kernel here")


# speedup vs baseline: 34.7607x; 34.7607x over previous
import functools

import jax
import jax.numpy as jnp
from jax import lax
from jax.experimental import pallas as pl
from jax.experimental.pallas import tpu as pltpu
from jax.experimental.pallas import tpu_sc as plsc

N_NODES = 10000
N_EDGES = 320000
D_FEAT = 128
NPAD = 10240
NTILES = 16
SLICE = NPAD // NTILES
EPT = N_EDGES // NTILES
EPASS = EPT // 2
NGROUPS = EPASS // 16
NCHUNK = SLICE // 16


def _matmul_tc(w1t_pad, xt):
    def body(w_ref, x_ref, o_ref):
        o_ref[...] = jax.lax.dot_general(
            w_ref[...], x_ref[...], (((1,), (0,)), ((), ())),
            preferred_element_type=jnp.float32)

    return pl.pallas_call(
        body,
        grid=(NPAD // 1024,),
        in_specs=[
            pl.BlockSpec((8, 128), lambda i: (0, 0)),
            pl.BlockSpec((128, 1024), lambda i: (0, i)),
        ],
        out_specs=pl.BlockSpec((8, 1024), lambda i: (0, i)),
        out_shape=jax.ShapeDtypeStruct((8, NPAD), jnp.float32),
    )(w1t_pad, xt)


def _tanh(v):
    return 2.0 / (1.0 + jnp.exp(-2.0 * v)) - 1.0


def _rsqrt(d):
    i = plsc.bitcast(d, jnp.int32)
    i = jnp.int32(0x5F3759DF) - (i >> 1)
    y = plsc.bitcast(i, jnp.float32)
    for _ in range(3):
        y = y * (1.5 - 0.5 * d * y * y)
    return y


def _sc_body(z1t, srch, dsth, w2h, w3h, wch, b1h, b2h, b3h, bch,
             out_o, hout_o, prt_o,
             A, B, dinvb, srcb, dstb, tmp, tmp2,
             w2v, w3v, wcv, b1v, b2v, b3v, bcv, sh1):
    s = lax.axis_index("s")
    base_n = s * SLICE
    base_e = s * EPT

    pltpu.sync_copy(w2h, w2v)
    pltpu.sync_copy(w3h, w3v)
    pltpu.sync_copy(wch, wcv)
    pltpu.sync_copy(b1h, b1v)
    pltpu.sync_copy(b2h, b2v)
    pltpu.sync_copy(b3h, b3v)
    pltpu.sync_copy(bch, bcv)

    def _zero_B():
        def zb(i, _):
            for f in range(4):
                B[f, pl.ds(i * 16, 16)] = jnp.zeros((16,), jnp.float32)
            return 0

        lax.fori_loop(0, NPAD // 16, zb, 0)

    def _reduce_B(F):
        pltpu.sync_copy(B, prt_o.at[s])
        plsc.subcore_barrier()
        pltpu.sync_copy(prt_o.at[0, :, pl.ds(base_n, SLICE)], tmp)
        for t in range(1, NTILES):
            pltpu.sync_copy(prt_o.at[t, :, pl.ds(base_n, SLICE)], tmp2)

            def acc_t(i, _):
                for f in range(F):
                    tmp[f, pl.ds(i * 16, 16)] = (
                        tmp[f, pl.ds(i * 16, 16)]
                        + tmp2[f, pl.ds(i * 16, 16)])
                return 0

            lax.fori_loop(0, NCHUNK, acc_t, 0)

    _zero_B()
    for p in range(2):
        pltpu.sync_copy(dsth.at[pl.ds(base_e + p * EPASS, EPASS)], dstb)

        def hg(g, _):
            dv = dstb[pl.ds(g * 16, 16)]
            cnt, last = plsc.scan_count(dv)
            plsc.addupdate_scatter(
                B, [jnp.zeros((16,), jnp.int32), dv],
                cnt.astype(jnp.float32), mask=last)
            return 0

        lax.fori_loop(0, NGROUPS, hg, 0)

    _reduce_B(1)

    def dv_(i, _):
        c = tmp[0, pl.ds(i * 16, 16)]
        tmp[1, pl.ds(i * 16, 16)] = _rsqrt(c + 1.0)
        return 0

    lax.fori_loop(0, NCHUNK, dv_, 0)
    pltpu.sync_copy(tmp.at[1], sh1.at[0, pl.ds(base_n, SLICE)])
    plsc.subcore_barrier()
    pltpu.sync_copy(sh1.at[0], dinvb)

    for f in range(4):
        pltpu.sync_copy(z1t.at[f], A.at[f])

    specs = [(4, None, b1v), (4, w2v, b2v), (2, w3v, b3v)]
    for F, wv, bv in specs:
        if wv is not None:
            for j in range(F):
                def mm(i, _, j=j):
                    o = base_n + i * 16
                    acc = jnp.zeros((16,), jnp.float32)
                    for ii in range(4):
                        acc = acc + wv[ii * F + j] * A[ii, pl.ds(o, 16)]
                    tmp[j, pl.ds(i * 16, 16)] = acc
                    return 0

                lax.fori_loop(0, NCHUNK, mm, 0)
            for j in range(F):
                pltpu.sync_copy(tmp.at[j], sh1.at[j, pl.ds(base_n, SLICE)])
            plsc.subcore_barrier()
            for j in range(F):
                pltpu.sync_copy(sh1.at[j], A.at[j])

        _zero_B()

        for p in range(2):
            pltpu.sync_copy(srch.at[pl.ds(base_e + p * EPASS, EPASS)], srcb)
            pltpu.sync_copy(dsth.at[pl.ds(base_e + p * EPASS, EPASS)], dstb)

            def eg(g, _):
                o = g * 16
                sv = srcb[pl.ds(o, 16)]
                dv = dstb[pl.ds(o, 16)]
                nrm = (plsc.load_gather(dinvb, [sv])
                       * plsc.load_gather(dinvb, [dv]))
                cnt, _ = plsc.scan_count(dv)
                fm = cnt == 1
                for f in range(F):
                    fs = jnp.full((16,), f, jnp.int32)
                    msg = plsc.load_gather(A, [fs, sv]) * nrm
                    plsc.addupdate_scatter(B, [fs, dv], msg, mask=fm)
                maxc = jnp.max(cnt)

                def dup(j, _):
                    m = cnt == j
                    for f in range(F):
                        fs = jnp.full((16,), f, jnp.int32)
                        msg = plsc.load_gather(A, [fs, sv]) * nrm
                        plsc.addupdate_scatter(B, [fs, dv], msg, mask=m)
                    return 0

                lax.fori_loop(2, maxc + 1, dup, 0)
                return 0

            lax.fori_loop(0, NGROUPS, eg, 0)

        _reduce_B(F)

        def post(i, _):
            o16 = i * 16
            on = base_n + o16
            di = dinvb[pl.ds(on, 16)]
            d2 = di * di
            for f in range(F):
                a = tmp[f, pl.ds(o16, 16)]
                z = A[f, pl.ds(on, 16)]
                tmp[f, pl.ds(o16, 16)] = _tanh(a + d2 * z + bv[f])
            return 0

        lax.fori_loop(0, NCHUNK, post, 0)
        for f in range(F):
            pltpu.sync_copy(tmp.at[f], sh1.at[f, pl.ds(base_n, SLICE)])
        plsc.subcore_barrier()
        for f in range(F):
            pltpu.sync_copy(sh1.at[f], A.at[f])
        plsc.subcore_barrier()

    def cls(i, _):
        o16 = i * 16
        on = base_n + o16
        h0 = A[0, pl.ds(on, 16)]
        h1 = A[1, pl.ds(on, 16)]
        tmp[0, pl.ds(o16, 16)] = wcv[0] * h0 + wcv[1] * h1 + bcv[0]
        return 0

    lax.fori_loop(0, NCHUNK, cls, 0)
    pltpu.sync_copy(tmp.at[0], out_o.at[pl.ds(base_n, SLICE)])
    for f in range(2):
        pltpu.sync_copy(A.at[f, pl.ds(base_n, SLICE)],
                        hout_o.at[f, pl.ds(base_n, SLICE)])


_sc_kernel = functools.partial(
    pl.kernel,
    out_type=[
        jax.ShapeDtypeStruct((NPAD,), jnp.float32),
        jax.ShapeDtypeStruct((2, NPAD), jnp.float32),
        jax.ShapeDtypeStruct((NTILES, 4, NPAD), jnp.float32),
    ],
    mesh=plsc.VectorSubcoreMesh(
        core_axis_name="c", subcore_axis_name="s", num_cores=1),
    compiler_params=pltpu.CompilerParams(needs_layout_passes=False),
    scratch_types=[
        pltpu.VMEM((4, NPAD), jnp.float32),
        pltpu.VMEM((4, NPAD), jnp.float32),
        pltpu.VMEM((NPAD,), jnp.float32),
        pltpu.VMEM((EPASS,), jnp.int32),
        pltpu.VMEM((EPASS,), jnp.int32),
        pltpu.VMEM((4, SLICE), jnp.float32),
        pltpu.VMEM((4, SLICE), jnp.float32),
        pltpu.VMEM((16, 16), jnp.float32),
        pltpu.VMEM((8, 16), jnp.float32),
        pltpu.VMEM((2, 16), jnp.float32),
        pltpu.VMEM((4, 16), jnp.float32),
        pltpu.VMEM((4, 16), jnp.float32),
        pltpu.VMEM((2, 16), jnp.float32),
        pltpu.VMEM((1, 16), jnp.float32),
        pltpu.VMEM_SHARED((4, NPAD), jnp.float32),
    ],
)(_sc_body)


def kernel(x, edge_index, W1, b1, W2, b2, W3, b3, Wc, bc):
    x_pad = jnp.pad(x, ((0, NPAD - N_NODES), (0, 0)))
    xt = x_pad.T
    w1t = jnp.zeros((8, D_FEAT), jnp.float32).at[:4].set(W1.T)
    z1t = _matmul_tc(w1t, xt)

    src = edge_index[0].astype(jnp.int32)
    dst = edge_index[1].astype(jnp.int32)

    w2sp = jnp.broadcast_to(W2.reshape(16, 1), (16, 16))
    w3sp = jnp.broadcast_to(W3.reshape(8, 1), (8, 16))
    wcsp = jnp.broadcast_to(Wc.reshape(2, 1), (2, 16))
    b1sp = jnp.broadcast_to(b1[:, None], (4, 16))
    b2sp = jnp.broadcast_to(b2[:, None], (4, 16))
    b3sp = jnp.broadcast_to(b3[:, None], (2, 16))
    bcsp = jnp.broadcast_to(bc[:, None], (1, 16))

    out_flat, hout, _ = _sc_kernel(z1t, src, dst, w2sp, w3sp, wcsp,
                                   b1sp, b2sp, b3sp, bcsp)
    out = out_flat[:N_NODES][:, None]
    h = hout[:, :N_NODES].T
    return out, h


# speedup vs baseline: 57.0044x; 1.6399x over previous
import functools

import jax
import jax.numpy as jnp
from jax import lax
from jax.experimental import pallas as pl
from jax.experimental.pallas import tpu as pltpu
from jax.experimental.pallas import tpu_sc as plsc

N_NODES = 10000
N_EDGES = 320000
D_FEAT = 128
NPAD = 10240
NTILES = 16
SLICE = NPAD // NTILES
EPT = N_EDGES // NTILES
EPASS = EPT // 2
NGROUPS = EPASS // 16
NCHUNK = SLICE // 16


def _matmul_tc(w1t_pad, xt):
    def body(w_ref, x_ref, o_ref):
        o_ref[...] = jax.lax.dot_general(
            w_ref[...], x_ref[...], (((1,), (0,)), ((), ())),
            preferred_element_type=jnp.float32)

    return pl.pallas_call(
        body,
        grid=(NPAD // 1024,),
        in_specs=[
            pl.BlockSpec((8, 128), lambda i: (0, 0)),
            pl.BlockSpec((128, 1024), lambda i: (0, i)),
        ],
        out_specs=pl.BlockSpec((8, 1024), lambda i: (0, i)),
        out_shape=jax.ShapeDtypeStruct((8, NPAD), jnp.float32),
    )(w1t_pad, xt)


def _tanh(v):
    return 2.0 / (1.0 + jnp.exp(-2.0 * v)) - 1.0


def _rsqrt(d):
    i = plsc.bitcast(d, jnp.int32)
    i = jnp.int32(0x5F3759DF) - (i >> 1)
    y = plsc.bitcast(i, jnp.float32)
    for _ in range(3):
        y = y * (1.5 - 0.5 * d * y * y)
    return y


def _sc_body(z1t, srch, dsth, wsh, bsh, wch, bch,
             out_o, hout_o, prt_o,
             A, B, dinvb, srcb, dstb, tmp, tmp2, tmp3,
             wsv, bsv, wcv, bcv, sh1, dma_sem, dma_sem2):
    s = lax.axis_index("s")
    base_n = s * SLICE
    base_e = s * EPT

    pltpu.sync_copy(wsh, wsv)
    pltpu.sync_copy(bsh, bsv)
    pltpu.sync_copy(wch, wcv)
    pltpu.sync_copy(bch, bcv)

    def _zero_B():
        @plsc.parallel_loop(0, NPAD // 16, unroll=8)
        def zb(i):
            for f in range(4):
                B[f, pl.ds(i * 16, 16)] = jnp.zeros((16,), jnp.float32)

    def _reduce_B():
        pltpu.sync_copy(B, prt_o.at[s])
        plsc.subcore_barrier()
        pltpu.sync_copy(prt_o.at[0, :, pl.ds(base_n, SLICE)], tmp)
        bufs = [tmp2, tmp3]
        sems = [dma_sem, dma_sem2]
        cps = [None, None]
        cps[1] = pltpu.async_copy(
            prt_o.at[1, :, pl.ds(base_n, SLICE)], bufs[1], sems[1])
        for t in range(1, NTILES):
            cur = bufs[t % 2]
            cps[t % 2].wait()
            if t + 1 < NTILES:
                cps[(t + 1) % 2] = pltpu.async_copy(
                    prt_o.at[t + 1, :, pl.ds(base_n, SLICE)],
                    bufs[(t + 1) % 2], sems[(t + 1) % 2])

            @plsc.parallel_loop(0, NCHUNK, unroll=4)
            def acc_t(i):
                for f in range(4):
                    tmp[f, pl.ds(i * 16, 16)] = (
                        tmp[f, pl.ds(i * 16, 16)]
                        + cur[f, pl.ds(i * 16, 16)])

    _zero_B()
    for p in range(2):
        pltpu.sync_copy(dsth.at[pl.ds(base_e + p * EPASS, EPASS)], dstb)

        @plsc.parallel_loop(0, NGROUPS, unroll=8)
        def hg(g):
            dv = dstb[pl.ds(g * 16, 16)]
            cnt, last = plsc.scan_count(dv)
            plsc.addupdate_scatter(
                B, [jnp.zeros((16,), jnp.int32), dv],
                cnt.astype(jnp.float32), mask=last)

    _reduce_B()

    @plsc.parallel_loop(0, NCHUNK, unroll=4)
    def dv_(i):
        c = tmp[0, pl.ds(i * 16, 16)]
        tmp[1, pl.ds(i * 16, 16)] = _rsqrt(c + 1.0)

    pltpu.sync_copy(tmp.at[1], sh1.at[0, pl.ds(base_n, SLICE)])
    plsc.subcore_barrier()
    pltpu.sync_copy(sh1.at[0], dinvb)

    for f in range(4):
        pltpu.sync_copy(z1t.at[f], A.at[f])

    def layer_body(l, _):
        @plsc.parallel_loop(0, NCHUNK, unroll=4)
        def mm(i):
            o = base_n + i * 16
            for j in range(4):
                acc = jnp.zeros((16,), jnp.float32)
                for ii in range(4):
                    acc = acc + wsv[l, ii * 4 + j] * A[ii, pl.ds(o, 16)]
                tmp[j, pl.ds(i * 16, 16)] = acc

        for j in range(4):
            pltpu.sync_copy(tmp.at[j], sh1.at[j, pl.ds(base_n, SLICE)])
        plsc.subcore_barrier()
        for j in range(4):
            pltpu.sync_copy(sh1.at[j], A.at[j])

        _zero_B()

        for p in range(2):
            pltpu.sync_copy(srch.at[pl.ds(base_e + p * EPASS, EPASS)], srcb)
            pltpu.sync_copy(dsth.at[pl.ds(base_e + p * EPASS, EPASS)], dstb)

            @plsc.parallel_loop(0, NGROUPS, unroll=8,
                                carry=jnp.zeros((16,), jnp.int32))
            def eg(g, cmax):
                o = g * 16
                sv = srcb[pl.ds(o, 16)]
                dv = dstb[pl.ds(o, 16)]
                nrm = (plsc.load_gather(dinvb, [sv])
                       * plsc.load_gather(dinvb, [dv]))
                cnt, _ = plsc.scan_count(dv)
                m1 = cnt == 1
                m2 = cnt == 2
                for f in range(4):
                    fs = jnp.full((16,), f, jnp.int32)
                    msg = plsc.load_gather(A, [fs, sv]) * nrm
                    plsc.addupdate_scatter(B, [fs, dv], msg, mask=m1)
                    plsc.addupdate_scatter(B, [fs, dv], msg, mask=m2)
                return jnp.maximum(cmax, cnt)

            ovf = jnp.max(eg)

            @pl.when(ovf > 2)
            def _fixup():
                def fg(g, _):
                    o = g * 16
                    sv = srcb[pl.ds(o, 16)]
                    dv = dstb[pl.ds(o, 16)]
                    nrm = (plsc.load_gather(dinvb, [sv])
                           * plsc.load_gather(dinvb, [dv]))
                    cnt, _ = plsc.scan_count(dv)
                    maxc = jnp.max(cnt)

                    def dup(j, __):
                        m = cnt == j
                        for f in range(4):
                            fs = jnp.full((16,), f, jnp.int32)
                            msg = plsc.load_gather(A, [fs, sv]) * nrm
                            plsc.addupdate_scatter(B, [fs, dv], msg,
                                                   mask=m)
                        return 0

                    lax.fori_loop(3, maxc + 1, dup, 0)
                    return 0

                lax.fori_loop(0, NGROUPS, fg, 0)

        _reduce_B()

        @plsc.parallel_loop(0, NCHUNK, unroll=4)
        def post(i):
            o16 = i * 16
            on = base_n + o16
            di = dinvb[pl.ds(on, 16)]
            d2 = di * di
            for f in range(4):
                a = tmp[f, pl.ds(o16, 16)]
                z = A[f, pl.ds(on, 16)]
                tmp[f, pl.ds(o16, 16)] = _tanh(a + d2 * z + bsv[l, f])

        for f in range(4):
            pltpu.sync_copy(tmp.at[f], sh1.at[f, pl.ds(base_n, SLICE)])
        plsc.subcore_barrier()
        for f in range(4):
            pltpu.sync_copy(sh1.at[f], A.at[f])
        plsc.subcore_barrier()
        return 0

    lax.fori_loop(0, 3, layer_body, 0)

    @plsc.parallel_loop(0, NCHUNK, unroll=4)
    def cls(i):
        o16 = i * 16
        on = base_n + o16
        h0 = A[0, pl.ds(on, 16)]
        h1 = A[1, pl.ds(on, 16)]
        tmp[0, pl.ds(o16, 16)] = wcv[0] * h0 + wcv[1] * h1 + bcv[0]

    pltpu.sync_copy(tmp.at[0], out_o.at[pl.ds(base_n, SLICE)])
    for f in range(2):
        pltpu.sync_copy(A.at[f, pl.ds(base_n, SLICE)],
                        hout_o.at[f, pl.ds(base_n, SLICE)])


_sc_kernel = functools.partial(
    pl.kernel,
    out_type=[
        jax.ShapeDtypeStruct((NPAD,), jnp.float32),
        jax.ShapeDtypeStruct((2, NPAD), jnp.float32),
        jax.ShapeDtypeStruct((NTILES, 4, NPAD), jnp.float32),
    ],
    mesh=plsc.VectorSubcoreMesh(
        core_axis_name="c", subcore_axis_name="s", num_cores=1),
    compiler_params=pltpu.CompilerParams(needs_layout_passes=False),
    scratch_types=[
        pltpu.VMEM((4, NPAD), jnp.float32),
        pltpu.VMEM((4, NPAD), jnp.float32),
        pltpu.VMEM((NPAD,), jnp.float32),
        pltpu.VMEM((EPASS,), jnp.int32),
        pltpu.VMEM((EPASS,), jnp.int32),
        pltpu.VMEM((4, SLICE), jnp.float32),
        pltpu.VMEM((4, SLICE), jnp.float32),
        pltpu.VMEM((4, SLICE), jnp.float32),
        pltpu.VMEM((3, 16, 16), jnp.float32),
        pltpu.VMEM((3, 4, 16), jnp.float32),
        pltpu.VMEM((2, 16), jnp.float32),
        pltpu.VMEM((1, 16), jnp.float32),
        pltpu.VMEM_SHARED((4, NPAD), jnp.float32),
        pltpu.SemaphoreType.DMA,
        pltpu.SemaphoreType.DMA,
    ],
)(_sc_body)


def kernel(x, edge_index, W1, b1, W2, b2, W3, b3, Wc, bc):
    x_pad = jnp.pad(x, ((0, NPAD - N_NODES), (0, 0)))
    xt = x_pad.T
    w1t = jnp.zeros((8, D_FEAT), jnp.float32).at[:4].set(W1.T)
    z1t = _matmul_tc(w1t, xt)

    src = edge_index[0].astype(jnp.int32)
    dst = edge_index[1].astype(jnp.int32)

    w3p = jnp.zeros((4, 4), jnp.float32).at[:, :2].set(W3)
    wstack = jnp.stack([jnp.eye(4, dtype=jnp.float32), W2, w3p])
    wsp = jnp.broadcast_to(wstack.reshape(3, 16, 1), (3, 16, 16))
    b3p = jnp.zeros((4,), jnp.float32).at[:2].set(b3)
    bstack = jnp.stack([b1, b2, b3p])
    bsp = jnp.broadcast_to(bstack.reshape(3, 4, 1), (3, 4, 16))
    wcsp = jnp.broadcast_to(Wc.reshape(2, 1), (2, 16))
    bcsp = jnp.broadcast_to(bc[:, None], (1, 16))

    out_flat, hout, _ = _sc_kernel(z1t, src, dst, wsp, bsp, wcsp, bcsp)
    out = out_flat[:N_NODES][:, None]
    h = hout[:, :N_NODES].T
    return out, h


# speedup vs baseline: 57.5146x; 1.0090x over previous
import functools

import jax
import jax.numpy as jnp
from jax import lax
from jax.experimental import pallas as pl
from jax.experimental.pallas import tpu as pltpu
from jax.experimental.pallas import tpu_sc as plsc

N_NODES = 10000
N_EDGES = 320000
D_FEAT = 128
NPAD = 10240
NTILES = 16
SLICE = NPAD // NTILES
EPT = N_EDGES // NTILES
EPASS = EPT // 2
NGROUPS = EPASS // 16
NCHUNK = SLICE // 16


def _matmul_tc(w1t_pad, x_pad):
    def body(w_ref, x_ref, o_ref):
        o_ref[...] = jax.lax.dot_general(
            w_ref[...], x_ref[...], (((1,), (1,)), ((), ())),
            preferred_element_type=jnp.float32)

    return pl.pallas_call(
        body,
        grid=(NPAD // 1024,),
        in_specs=[
            pl.BlockSpec((8, D_FEAT), lambda i: (0, 0)),
            pl.BlockSpec((1024, D_FEAT), lambda i: (i, 0)),
        ],
        out_specs=pl.BlockSpec((8, 1024), lambda i: (0, i)),
        out_shape=jax.ShapeDtypeStruct((8, NPAD), jnp.float32),
    )(w1t_pad, x_pad)


def _tanh(v):
    return 2.0 / (1.0 + jnp.exp(-2.0 * v)) - 1.0


def _rsqrt(d):
    i = plsc.bitcast(d, jnp.int32)
    i = jnp.int32(0x5F3759DF) - (i >> 1)
    y = plsc.bitcast(i, jnp.float32)
    for _ in range(3):
        y = y * (1.5 - 0.5 * d * y * y)
    return y


def _sc_body(z1t, srch, dsth, wsh, bsh, wch, bch,
             out_o, hout_o, prt_o,
             A, B, dinvb, srcb, dstb, tmp, tmp2, tmp3,
             wsv, bsv, wcv, bcv, sh1, dma_sem, dma_sem2):
    s = lax.axis_index("s")
    base_n = s * SLICE
    base_e = s * EPT

    pltpu.sync_copy(wsh, wsv)
    pltpu.sync_copy(bsh, bsv)
    pltpu.sync_copy(wch, wcv)
    pltpu.sync_copy(bch, bcv)

    def _zero_B():
        @plsc.parallel_loop(0, NPAD // 16, unroll=8)
        def zb(i):
            for f in range(4):
                B[f, pl.ds(i * 16, 16)] = jnp.zeros((16,), jnp.float32)

    def _reduce_B():
        pltpu.sync_copy(B, prt_o.at[s])
        plsc.subcore_barrier()
        pltpu.sync_copy(prt_o.at[0, :, pl.ds(base_n, SLICE)], tmp)
        bufs = [tmp2, tmp3]
        sems = [dma_sem, dma_sem2]
        cps = [None, None]
        cps[1] = pltpu.async_copy(
            prt_o.at[1, :, pl.ds(base_n, SLICE)], bufs[1], sems[1])
        for t in range(1, NTILES):
            cur = bufs[t % 2]
            cps[t % 2].wait()
            if t + 1 < NTILES:
                cps[(t + 1) % 2] = pltpu.async_copy(
                    prt_o.at[t + 1, :, pl.ds(base_n, SLICE)],
                    bufs[(t + 1) % 2], sems[(t + 1) % 2])

            @plsc.parallel_loop(0, NCHUNK, unroll=4)
            def acc_t(i):
                for f in range(4):
                    tmp[f, pl.ds(i * 16, 16)] = (
                        tmp[f, pl.ds(i * 16, 16)]
                        + cur[f, pl.ds(i * 16, 16)])

    _zero_B()
    for p in range(2):
        pltpu.sync_copy(dsth.at[pl.ds(base_e + p * EPASS, EPASS)], dstb)

        @plsc.parallel_loop(0, NGROUPS, unroll=8)
        def hg(g):
            dv = dstb[pl.ds(g * 16, 16)]
            cnt, last = plsc.scan_count(dv)
            plsc.addupdate_scatter(
                B, [jnp.zeros((16,), jnp.int32), dv],
                cnt.astype(jnp.float32), mask=last)

    _reduce_B()

    @plsc.parallel_loop(0, NCHUNK, unroll=4)
    def dv_(i):
        c = tmp[0, pl.ds(i * 16, 16)]
        tmp[1, pl.ds(i * 16, 16)] = _rsqrt(c + 1.0)

    pltpu.sync_copy(tmp.at[1], sh1.at[0, pl.ds(base_n, SLICE)])
    plsc.subcore_barrier()
    pltpu.sync_copy(sh1.at[0], dinvb)

    for f in range(4):
        pltpu.sync_copy(z1t.at[f], A.at[f])

    def layer_body(l, _):
        @plsc.parallel_loop(0, NCHUNK, unroll=4)
        def mm(i):
            o = base_n + i * 16
            for j in range(4):
                acc = jnp.zeros((16,), jnp.float32)
                for ii in range(4):
                    acc = acc + wsv[l, ii * 4 + j] * A[ii, pl.ds(o, 16)]
                tmp[j, pl.ds(i * 16, 16)] = acc

        for j in range(4):
            pltpu.sync_copy(tmp.at[j], sh1.at[j, pl.ds(base_n, SLICE)])
        plsc.subcore_barrier()
        for j in range(4):
            pltpu.sync_copy(sh1.at[j], A.at[j])

        _zero_B()

        for p in range(2):
            pltpu.sync_copy(srch.at[pl.ds(base_e + p * EPASS, EPASS)], srcb)
            pltpu.sync_copy(dsth.at[pl.ds(base_e + p * EPASS, EPASS)], dstb)

            @plsc.parallel_loop(0, NGROUPS, unroll=8,
                                carry=jnp.zeros((16,), jnp.int32))
            def eg(g, cmax):
                o = g * 16
                sv = srcb[pl.ds(o, 16)]
                dv = dstb[pl.ds(o, 16)]
                nrm = (plsc.load_gather(dinvb, [sv])
                       * plsc.load_gather(dinvb, [dv]))
                cnt, _ = plsc.scan_count(dv)
                m1 = cnt == 1
                m2 = cnt == 2
                for f in range(4):
                    fs = jnp.full((16,), f, jnp.int32)
                    msg = plsc.load_gather(A, [fs, sv]) * nrm
                    plsc.addupdate_scatter(B, [fs, dv], msg, mask=m1)
                    plsc.addupdate_scatter(B, [fs, dv], msg, mask=m2)
                return jnp.maximum(cmax, cnt)

            ovf = jnp.max(eg)

            @pl.when(ovf > 2)
            def _fixup():
                def fg(g, _):
                    o = g * 16
                    sv = srcb[pl.ds(o, 16)]
                    dv = dstb[pl.ds(o, 16)]
                    nrm = (plsc.load_gather(dinvb, [sv])
                           * plsc.load_gather(dinvb, [dv]))
                    cnt, _ = plsc.scan_count(dv)
                    maxc = jnp.max(cnt)

                    def dup(j, __):
                        m = cnt == j
                        for f in range(4):
                            fs = jnp.full((16,), f, jnp.int32)
                            msg = plsc.load_gather(A, [fs, sv]) * nrm
                            plsc.addupdate_scatter(B, [fs, dv], msg,
                                                   mask=m)
                        return 0

                    lax.fori_loop(3, maxc + 1, dup, 0)
                    return 0

                lax.fori_loop(0, NGROUPS, fg, 0)

        _reduce_B()

        @plsc.parallel_loop(0, NCHUNK, unroll=4)
        def post(i):
            o16 = i * 16
            on = base_n + o16
            di = dinvb[pl.ds(on, 16)]
            d2 = di * di
            for f in range(4):
                a = tmp[f, pl.ds(o16, 16)]
                z = A[f, pl.ds(on, 16)]
                tmp[f, pl.ds(o16, 16)] = _tanh(a + d2 * z + bsv[l, f])

        for f in range(4):
            pltpu.sync_copy(tmp.at[f], sh1.at[f, pl.ds(base_n, SLICE)])
        plsc.subcore_barrier()
        for f in range(4):
            pltpu.sync_copy(sh1.at[f], A.at[f])
        plsc.subcore_barrier()
        return 0

    lax.fori_loop(0, 3, layer_body, 0)

    @plsc.parallel_loop(0, NCHUNK, unroll=4)
    def cls(i):
        o16 = i * 16
        on = base_n + o16
        h0 = A[0, pl.ds(on, 16)]
        h1 = A[1, pl.ds(on, 16)]
        tmp[0, pl.ds(o16, 16)] = wcv[0] * h0 + wcv[1] * h1 + bcv[0]

    pltpu.sync_copy(tmp.at[0], out_o.at[pl.ds(base_n, SLICE)])
    for f in range(2):
        pltpu.sync_copy(A.at[f, pl.ds(base_n, SLICE)],
                        hout_o.at[f, pl.ds(base_n, SLICE)])


_sc_kernel = functools.partial(
    pl.kernel,
    out_type=[
        jax.ShapeDtypeStruct((NPAD,), jnp.float32),
        jax.ShapeDtypeStruct((2, NPAD), jnp.float32),
        jax.ShapeDtypeStruct((NTILES, 4, NPAD), jnp.float32),
    ],
    mesh=plsc.VectorSubcoreMesh(
        core_axis_name="c", subcore_axis_name="s", num_cores=1),
    compiler_params=pltpu.CompilerParams(needs_layout_passes=False),
    scratch_types=[
        pltpu.VMEM((4, NPAD), jnp.float32),
        pltpu.VMEM((4, NPAD), jnp.float32),
        pltpu.VMEM((NPAD,), jnp.float32),
        pltpu.VMEM((EPASS,), jnp.int32),
        pltpu.VMEM((EPASS,), jnp.int32),
        pltpu.VMEM((4, SLICE), jnp.float32),
        pltpu.VMEM((4, SLICE), jnp.float32),
        pltpu.VMEM((4, SLICE), jnp.float32),
        pltpu.VMEM((3, 16, 16), jnp.float32),
        pltpu.VMEM((3, 4, 16), jnp.float32),
        pltpu.VMEM((2, 16), jnp.float32),
        pltpu.VMEM((1, 16), jnp.float32),
        pltpu.VMEM_SHARED((4, NPAD), jnp.float32),
        pltpu.SemaphoreType.DMA,
        pltpu.SemaphoreType.DMA,
    ],
)(_sc_body)


def kernel(x, edge_index, W1, b1, W2, b2, W3, b3, Wc, bc):
    x_pad = jnp.pad(x, ((0, NPAD - N_NODES), (0, 0)))
    w1t = jnp.zeros((8, D_FEAT), jnp.float32).at[:4].set(W1.T)
    z1t = _matmul_tc(w1t, x_pad)

    src = edge_index[0].astype(jnp.int32)
    dst = edge_index[1].astype(jnp.int32)

    w3p = jnp.zeros((4, 4), jnp.float32).at[:, :2].set(W3)
    wstack = jnp.stack([jnp.eye(4, dtype=jnp.float32), W2, w3p])
    wsp = jnp.broadcast_to(wstack.reshape(3, 16, 1), (3, 16, 16))
    b3p = jnp.zeros((4,), jnp.float32).at[:2].set(b3)
    bstack = jnp.stack([b1, b2, b3p])
    bsp = jnp.broadcast_to(bstack.reshape(3, 4, 1), (3, 4, 16))
    wcsp = jnp.broadcast_to(Wc.reshape(2, 1), (2, 16))
    bcsp = jnp.broadcast_to(bc[:, None], (1, 16))

    out_flat, hout, _ = _sc_kernel(z1t, src, dst, wsp, bsp, wcsp, bcsp)
    out = out_flat[:N_NODES][:, None]
    h = hout[:, :N_NODES].T
    return out, h


# speedup vs baseline: 57.5243x; 1.0002x over previous
import functools

import jax
import jax.numpy as jnp
from jax import lax
from jax.experimental import pallas as pl
from jax.experimental.pallas import tpu as pltpu
from jax.experimental.pallas import tpu_sc as plsc

N_NODES = 10000
N_EDGES = 320000
D_FEAT = 128
NPAD = 10240
NTILES = 16
SLICE = NPAD // NTILES
EPT = N_EDGES // NTILES
EPASS = EPT // 2
NGROUPS = EPASS // 16
NCHUNK = SLICE // 16


def _matmul_tc(w1t_pad, x_pad):
    def body(w_ref, x_ref, o_ref):
        o_ref[...] = jax.lax.dot_general(
            w_ref[...], x_ref[...], (((1,), (1,)), ((), ())),
            preferred_element_type=jnp.float32)

    return pl.pallas_call(
        body,
        grid=(NPAD // 1024,),
        in_specs=[
            pl.BlockSpec((8, D_FEAT), lambda i: (0, 0)),
            pl.BlockSpec((1024, D_FEAT), lambda i: (i, 0)),
        ],
        out_specs=pl.BlockSpec((8, 1024), lambda i: (0, i)),
        out_shape=jax.ShapeDtypeStruct((8, NPAD), jnp.float32),
    )(w1t_pad, x_pad)


def _tanh(v):
    return 2.0 / (1.0 + jnp.exp(-2.0 * v)) - 1.0


def _rsqrt(d):
    i = plsc.bitcast(d, jnp.int32)
    i = jnp.int32(0x5F3759DF) - (i >> 1)
    y = plsc.bitcast(i, jnp.float32)
    for _ in range(3):
        y = y * (1.5 - 0.5 * d * y * y)
    return y


def _sc_body(z1t, srch, dsth, wsh, bsh, wch, bch,
             out_o, hout_o, prt_o,
             A, B, dinvb, srcb, dstb, tmp, tmp2, tmp3,
             wsv, bsv, wcv, bcv, sh1, dma_sem, dma_sem2):
    s = lax.axis_index("s")
    base_n = s * SLICE
    base_e = s * EPT

    pltpu.sync_copy(wsh, wsv)
    pltpu.sync_copy(bsh, bsv)
    pltpu.sync_copy(wch, wcv)
    pltpu.sync_copy(bch, bcv)

    def _zero_B():
        @plsc.parallel_loop(0, NPAD // 16, unroll=8)
        def zb(i):
            for f in range(4):
                B[f, pl.ds(i * 16, 16)] = jnp.zeros((16,), jnp.float32)

    def _reduce_B():
        with jax.named_scope("reduceB"):
            pltpu.sync_copy(B, prt_o.at[s])
            plsc.subcore_barrier()
            pltpu.sync_copy(prt_o.at[0, :, pl.ds(base_n, SLICE)], tmp)
            bufs = [tmp2, tmp3]
            sems = [dma_sem, dma_sem2]
            cps = [None, None]
            cps[1] = pltpu.async_copy(
                prt_o.at[1, :, pl.ds(base_n, SLICE)], bufs[1], sems[1])
            for t in range(1, NTILES):
                cur = bufs[t % 2]
                cps[t % 2].wait()
                if t + 1 < NTILES:
                    cps[(t + 1) % 2] = pltpu.async_copy(
                        prt_o.at[t + 1, :, pl.ds(base_n, SLICE)],
                        bufs[(t + 1) % 2], sems[(t + 1) % 2])

                @plsc.parallel_loop(0, NCHUNK, unroll=4)
                def acc_t(i):
                    for f in range(4):
                        tmp[f, pl.ds(i * 16, 16)] = (
                            tmp[f, pl.ds(i * 16, 16)]
                            + cur[f, pl.ds(i * 16, 16)])

    with jax.named_scope("hist"):
        _zero_B()
        for p in range(2):
            pltpu.sync_copy(dsth.at[pl.ds(base_e + p * EPASS, EPASS)],
                            dstb)

            @plsc.parallel_loop(0, NGROUPS, unroll=8)
            def hg(g):
                dv = dstb[pl.ds(g * 16, 16)]
                cnt, last = plsc.scan_count(dv)
                plsc.addupdate_scatter(
                    B, [jnp.zeros((16,), jnp.int32), dv],
                    cnt.astype(jnp.float32), mask=last)

    _reduce_B()

    with jax.named_scope("dinv"):
        @plsc.parallel_loop(0, NCHUNK, unroll=4)
        def dv_(i):
            c = tmp[0, pl.ds(i * 16, 16)]
            tmp[1, pl.ds(i * 16, 16)] = _rsqrt(c + 1.0)

        pltpu.sync_copy(tmp.at[1], sh1.at[0, pl.ds(base_n, SLICE)])
        plsc.subcore_barrier()
        pltpu.sync_copy(sh1.at[0], dinvb)

    with jax.named_scope("loadz1"):
        for f in range(4):
            pltpu.sync_copy(z1t.at[f], A.at[f])

    def layer_body(l, _):
        with jax.named_scope("mm"):
            @plsc.parallel_loop(0, NCHUNK, unroll=4)
            def mm(i):
                o = base_n + i * 16
                for j in range(4):
                    acc = jnp.zeros((16,), jnp.float32)
                    for ii in range(4):
                        acc = (acc
                               + wsv[l, ii * 4 + j] * A[ii, pl.ds(o, 16)])
                    tmp[j, pl.ds(i * 16, 16)] = acc

            for j in range(4):
                pltpu.sync_copy(tmp.at[j], sh1.at[j, pl.ds(base_n, SLICE)])
            plsc.subcore_barrier()
            for j in range(4):
                pltpu.sync_copy(sh1.at[j], A.at[j])

        with jax.named_scope("zeroB"):
            _zero_B()

        for p in range(2):
            with jax.named_scope("edgeload"):
                pltpu.sync_copy(srch.at[pl.ds(base_e + p * EPASS, EPASS)],
                                srcb)
                pltpu.sync_copy(dsth.at[pl.ds(base_e + p * EPASS, EPASS)],
                                dstb)

            with jax.named_scope("edges"):
                @plsc.parallel_loop(0, NGROUPS, unroll=8,
                                    carry=jnp.zeros((16,), jnp.int32))
                def eg(g, cmax):
                    o = g * 16
                    sv = srcb[pl.ds(o, 16)]
                    dv = dstb[pl.ds(o, 16)]
                    nrm = (plsc.load_gather(dinvb, [sv])
                           * plsc.load_gather(dinvb, [dv]))
                    cnt, _ = plsc.scan_count(dv)
                    m1 = cnt == 1
                    m2 = cnt == 2
                    for f in range(4):
                        fs = jnp.full((16,), f, jnp.int32)
                        msg = plsc.load_gather(A, [fs, sv]) * nrm
                        plsc.addupdate_scatter(B, [fs, dv], msg, mask=m1)
                        plsc.addupdate_scatter(B, [fs, dv], msg, mask=m2)
                    return jnp.maximum(cmax, cnt)

                ovf = jnp.max(eg)

                @pl.when(ovf > 2)
                def _fixup():
                    def fg(g, _):
                        o = g * 16
                        sv = srcb[pl.ds(o, 16)]
                        dv = dstb[pl.ds(o, 16)]
                        nrm = (plsc.load_gather(dinvb, [sv])
                               * plsc.load_gather(dinvb, [dv]))
                        cnt, _ = plsc.scan_count(dv)
                        maxc = jnp.max(cnt)

                        def dup(j, __):
                            m = cnt == j
                            for f in range(4):
                                fs = jnp.full((16,), f, jnp.int32)
                                msg = plsc.load_gather(A, [fs, sv]) * nrm
                                plsc.addupdate_scatter(B, [fs, dv], msg,
                                                       mask=m)
                            return 0

                        lax.fori_loop(3, maxc + 1, dup, 0)
                        return 0

                    lax.fori_loop(0, NGROUPS, fg, 0)

        _reduce_B()

        with jax.named_scope("post"):
            @plsc.parallel_loop(0, NCHUNK, unroll=4)
            def post(i):
                o16 = i * 16
                on = base_n + o16
                di = dinvb[pl.ds(on, 16)]
                d2 = di * di
                for f in range(4):
                    a = tmp[f, pl.ds(o16, 16)]
                    z = A[f, pl.ds(on, 16)]
                    tmp[f, pl.ds(o16, 16)] = _tanh(a + d2 * z + bsv[l, f])

            for f in range(4):
                pltpu.sync_copy(tmp.at[f], sh1.at[f, pl.ds(base_n, SLICE)])
            plsc.subcore_barrier()
            for f in range(4):
                pltpu.sync_copy(sh1.at[f], A.at[f])
            plsc.subcore_barrier()
        return 0

    lax.fori_loop(0, 3, layer_body, 0)

    @plsc.parallel_loop(0, NCHUNK, unroll=4)
    def cls(i):
        o16 = i * 16
        on = base_n + o16
        h0 = A[0, pl.ds(on, 16)]
        h1 = A[1, pl.ds(on, 16)]
        tmp[0, pl.ds(o16, 16)] = wcv[0] * h0 + wcv[1] * h1 + bcv[0]

    pltpu.sync_copy(tmp.at[0], out_o.at[pl.ds(base_n, SLICE)])
    for f in range(2):
        pltpu.sync_copy(A.at[f, pl.ds(base_n, SLICE)],
                        hout_o.at[f, pl.ds(base_n, SLICE)])


_sc_kernel = functools.partial(
    pl.kernel,
    out_type=[
        jax.ShapeDtypeStruct((NPAD,), jnp.float32),
        jax.ShapeDtypeStruct((2, NPAD), jnp.float32),
        jax.ShapeDtypeStruct((NTILES, 4, NPAD), jnp.float32),
    ],
    mesh=plsc.VectorSubcoreMesh(
        core_axis_name="c", subcore_axis_name="s", num_cores=1),
    compiler_params=pltpu.CompilerParams(needs_layout_passes=False),
    scratch_types=[
        pltpu.VMEM((4, NPAD), jnp.float32),
        pltpu.VMEM((4, NPAD), jnp.float32),
        pltpu.VMEM((NPAD,), jnp.float32),
        pltpu.VMEM((EPASS,), jnp.int32),
        pltpu.VMEM((EPASS,), jnp.int32),
        pltpu.VMEM((4, SLICE), jnp.float32),
        pltpu.VMEM((4, SLICE), jnp.float32),
        pltpu.VMEM((4, SLICE), jnp.float32),
        pltpu.VMEM((3, 16, 16), jnp.float32),
        pltpu.VMEM((3, 4, 16), jnp.float32),
        pltpu.VMEM((2, 16), jnp.float32),
        pltpu.VMEM((1, 16), jnp.float32),
        pltpu.VMEM_SHARED((4, NPAD), jnp.float32),
        pltpu.SemaphoreType.DMA,
        pltpu.SemaphoreType.DMA,
    ],
)(_sc_body)


def kernel(x, edge_index, W1, b1, W2, b2, W3, b3, Wc, bc):
    x_pad = jnp.pad(x, ((0, NPAD - N_NODES), (0, 0)))
    w1t = jnp.zeros((8, D_FEAT), jnp.float32).at[:4].set(W1.T)
    z1t = _matmul_tc(w1t, x_pad)

    src = edge_index[0].astype(jnp.int32)
    dst = edge_index[1].astype(jnp.int32)

    w3p = jnp.zeros((4, 4), jnp.float32).at[:, :2].set(W3)
    wstack = jnp.stack([jnp.eye(4, dtype=jnp.float32), W2, w3p])
    wsp = jnp.broadcast_to(wstack.reshape(3, 16, 1), (3, 16, 16))
    b3p = jnp.zeros((4,), jnp.float32).at[:2].set(b3)
    bstack = jnp.stack([b1, b2, b3p])
    bsp = jnp.broadcast_to(bstack.reshape(3, 4, 1), (3, 4, 16))
    wcsp = jnp.broadcast_to(Wc.reshape(2, 1), (2, 16))
    bcsp = jnp.broadcast_to(bc[:, None], (1, 16))

    out_flat, hout, _ = _sc_kernel(z1t, src, dst, wsp, bsp, wcsp, bcsp)
    out = out_flat[:N_NODES][:, None]
    h = hout[:, :N_NODES].T
    return out, h


# speedup vs baseline: 58.8322x; 1.0227x over previous
import functools

import jax
import jax.numpy as jnp
from jax import lax
from jax.experimental import pallas as pl
from jax.experimental.pallas import tpu as pltpu
from jax.experimental.pallas import tpu_sc as plsc

N_NODES = 10000
N_EDGES = 320000
D_FEAT = 128
NPAD = 10240
NTILES = 16
SLICE = NPAD // NTILES
EPT = N_EDGES // NTILES
EPASS = EPT // 2
NGROUPS = EPASS // 16
NCHUNK = SLICE // 16


def _matmul_tc(w1t_pad, x):
    def body(w_ref, x_ref, o_ref):
        o_ref[...] = jax.lax.dot_general(
            w_ref[...], x_ref[...], (((1,), (1,)), ((), ())),
            preferred_element_type=jnp.float32)

    return pl.pallas_call(
        body,
        grid=(NPAD // 1024,),
        in_specs=[
            pl.BlockSpec((8, D_FEAT), lambda i: (0, 0)),
            pl.BlockSpec((1024, D_FEAT), lambda i: (i, 0)),
        ],
        out_specs=pl.BlockSpec((8, 1024), lambda i: (0, i)),
        out_shape=jax.ShapeDtypeStruct((8, NPAD), jnp.float32),
    )(w1t_pad, x)


def _tanh(v):
    return 2.0 / (1.0 + jnp.exp(-2.0 * v)) - 1.0


def _rsqrt(d):
    i = plsc.bitcast(d, jnp.int32)
    i = jnp.int32(0x5F3759DF) - (i >> 1)
    y = plsc.bitcast(i, jnp.float32)
    for _ in range(3):
        y = y * (1.5 - 0.5 * d * y * y)
    return y


def _sc_body(z1t, srch, dsth, cfg,
             out_o, hout_o, prt_o,
             A, B, dinvb, srcb, dstb, tmp, tmp2, tmp3,
             cfgv, sh1, dma_sem, dma_sem2):
    s = lax.axis_index("s")
    base_n = s * SLICE
    base_e = s * EPT

    pltpu.sync_copy(cfg, cfgv)

    def _zero_B():
        @plsc.parallel_loop(0, NPAD // 16, unroll=8)
        def zb(i):
            for f in range(4):
                B[f, pl.ds(i * 16, 16)] = jnp.zeros((16,), jnp.float32)

    def _reduce_B():
        with jax.named_scope("reduceB"):
            pltpu.sync_copy(B, prt_o.at[s])
            plsc.subcore_barrier()
            pltpu.sync_copy(prt_o.at[0, :, pl.ds(base_n, SLICE)], tmp)
            bufs = [tmp2, tmp3]
            sems = [dma_sem, dma_sem2]
            cps = [None, None]
            cps[1] = pltpu.async_copy(
                prt_o.at[1, :, pl.ds(base_n, SLICE)], bufs[1], sems[1])
            for t in range(1, NTILES):
                cur = bufs[t % 2]
                cps[t % 2].wait()
                if t + 1 < NTILES:
                    cps[(t + 1) % 2] = pltpu.async_copy(
                        prt_o.at[t + 1, :, pl.ds(base_n, SLICE)],
                        bufs[(t + 1) % 2], sems[(t + 1) % 2])

                @plsc.parallel_loop(0, NCHUNK, unroll=4)
                def acc_t(i):
                    for f in range(4):
                        tmp[f, pl.ds(i * 16, 16)] = (
                            tmp[f, pl.ds(i * 16, 16)]
                            + cur[f, pl.ds(i * 16, 16)])

    with jax.named_scope("hist"):
        _zero_B()
        for p in range(2):
            pltpu.sync_copy(dsth.at[pl.ds(base_e + p * EPASS, EPASS)],
                            dstb)

            @plsc.parallel_loop(0, NGROUPS, unroll=8)
            def hg(g):
                dv = dstb[pl.ds(g * 16, 16)]
                cnt, last = plsc.scan_count(dv)
                plsc.addupdate_scatter(
                    B, [jnp.zeros((16,), jnp.int32), dv],
                    cnt.astype(jnp.float32), mask=last)

    _reduce_B()

    with jax.named_scope("dinv"):
        @plsc.parallel_loop(0, NCHUNK, unroll=4)
        def dv_(i):
            c = tmp[0, pl.ds(i * 16, 16)]
            tmp[1, pl.ds(i * 16, 16)] = _rsqrt(c + 1.0)

        pltpu.sync_copy(tmp.at[1], sh1.at[0, pl.ds(base_n, SLICE)])
        plsc.subcore_barrier()
        pltpu.sync_copy(sh1.at[0], dinvb)

    with jax.named_scope("loadz1"):
        for f in range(4):
            pltpu.sync_copy(z1t.at[f], A.at[f])

    def layer_body(l, _):
        with jax.named_scope("mm"):
            @plsc.parallel_loop(0, NCHUNK, unroll=4)
            def mm(i):
                o = base_n + i * 16
                for j in range(4):
                    acc = jnp.zeros((16,), jnp.float32)
                    for ii in range(4):
                        acc = (acc
                               + cfgv[l * 16 + ii * 4 + j] * A[ii, pl.ds(o, 16)])
                    tmp[j, pl.ds(i * 16, 16)] = acc

            for j in range(4):
                pltpu.sync_copy(tmp.at[j], sh1.at[j, pl.ds(base_n, SLICE)])
            plsc.subcore_barrier()
            for j in range(4):
                pltpu.sync_copy(sh1.at[j], A.at[j])

        with jax.named_scope("zeroB"):
            _zero_B()

        for p in range(2):
            with jax.named_scope("edgeload"):
                pltpu.sync_copy(srch.at[pl.ds(base_e + p * EPASS, EPASS)],
                                srcb)
                pltpu.sync_copy(dsth.at[pl.ds(base_e + p * EPASS, EPASS)],
                                dstb)

            with jax.named_scope("edges"):
                @plsc.parallel_loop(0, NGROUPS, unroll=8,
                                    carry=jnp.zeros((16,), jnp.int32))
                def eg(g, cmax):
                    o = g * 16
                    sv = srcb[pl.ds(o, 16)]
                    dv = dstb[pl.ds(o, 16)]
                    nrm = (plsc.load_gather(dinvb, [sv])
                           * plsc.load_gather(dinvb, [dv]))
                    cnt, _ = plsc.scan_count(dv)
                    m1 = cnt == 1
                    m2 = cnt == 2
                    for f in range(4):
                        fs = jnp.full((16,), f, jnp.int32)
                        msg = plsc.load_gather(A, [fs, sv]) * nrm
                        plsc.addupdate_scatter(B, [fs, dv], msg, mask=m1)
                        plsc.addupdate_scatter(B, [fs, dv], msg, mask=m2)
                    return jnp.maximum(cmax, cnt)

                ovf = jnp.max(eg)

                @pl.when(ovf > 2)
                def _fixup():
                    def fg(g, _):
                        o = g * 16
                        sv = srcb[pl.ds(o, 16)]
                        dv = dstb[pl.ds(o, 16)]
                        nrm = (plsc.load_gather(dinvb, [sv])
                               * plsc.load_gather(dinvb, [dv]))
                        cnt, _ = plsc.scan_count(dv)
                        maxc = jnp.max(cnt)

                        def dup(j, __):
                            m = cnt == j
                            for f in range(4):
                                fs = jnp.full((16,), f, jnp.int32)
                                msg = plsc.load_gather(A, [fs, sv]) * nrm
                                plsc.addupdate_scatter(B, [fs, dv], msg,
                                                       mask=m)
                            return 0

                        lax.fori_loop(3, maxc + 1, dup, 0)
                        return 0

                    lax.fori_loop(0, NGROUPS, fg, 0)

        _reduce_B()

        with jax.named_scope("post"):
            @plsc.parallel_loop(0, NCHUNK, unroll=4)
            def post(i):
                o16 = i * 16
                on = base_n + o16
                di = dinvb[pl.ds(on, 16)]
                d2 = di * di
                for f in range(4):
                    a = tmp[f, pl.ds(o16, 16)]
                    z = A[f, pl.ds(on, 16)]
                    tmp[f, pl.ds(o16, 16)] = _tanh(a + d2 * z + cfgv[48 + l * 4 + f])

            for f in range(4):
                pltpu.sync_copy(tmp.at[f], sh1.at[f, pl.ds(base_n, SLICE)])
            plsc.subcore_barrier()
            for f in range(4):
                pltpu.sync_copy(sh1.at[f], A.at[f])
            plsc.subcore_barrier()
        return 0

    lax.fori_loop(0, 3, layer_body, 0)

    @plsc.parallel_loop(0, NCHUNK, unroll=4)
    def cls(i):
        o16 = i * 16
        on = base_n + o16
        h0 = A[0, pl.ds(on, 16)]
        h1 = A[1, pl.ds(on, 16)]
        tmp[0, pl.ds(o16, 16)] = cfgv[60] * h0 + cfgv[61] * h1 + cfgv[62]

    pltpu.sync_copy(tmp.at[0], out_o.at[pl.ds(base_n, SLICE)])
    for f in range(2):
        pltpu.sync_copy(A.at[f, pl.ds(base_n, SLICE)],
                        hout_o.at[f, pl.ds(base_n, SLICE)])


_sc_kernel = functools.partial(
    pl.kernel,
    out_type=[
        jax.ShapeDtypeStruct((NPAD,), jnp.float32),
        jax.ShapeDtypeStruct((2, NPAD), jnp.float32),
        jax.ShapeDtypeStruct((NTILES, 4, NPAD), jnp.float32),
    ],
    mesh=plsc.VectorSubcoreMesh(
        core_axis_name="c", subcore_axis_name="s", num_cores=1),
    compiler_params=pltpu.CompilerParams(needs_layout_passes=False),
    scratch_types=[
        pltpu.VMEM((4, NPAD), jnp.float32),
        pltpu.VMEM((4, NPAD), jnp.float32),
        pltpu.VMEM((NPAD,), jnp.float32),
        pltpu.VMEM((EPASS,), jnp.int32),
        pltpu.VMEM((EPASS,), jnp.int32),
        pltpu.VMEM((4, SLICE), jnp.float32),
        pltpu.VMEM((4, SLICE), jnp.float32),
        pltpu.VMEM((4, SLICE), jnp.float32),
        pltpu.VMEM((63, 16), jnp.float32),
        pltpu.VMEM_SHARED((4, NPAD), jnp.float32),
        pltpu.SemaphoreType.DMA,
        pltpu.SemaphoreType.DMA,
    ],
)(_sc_body)


def kernel(x, edge_index, W1, b1, W2, b2, W3, b3, Wc, bc):
    w1t = jnp.zeros((8, D_FEAT), jnp.float32).at[:4].set(W1.T)
    z1t = _matmul_tc(w1t, x)

    src = edge_index[0].astype(jnp.int32)
    dst = edge_index[1].astype(jnp.int32)

    w3p = jnp.zeros((4, 4), jnp.float32).at[:, :2].set(W3)
    wstack = jnp.stack([jnp.eye(4, dtype=jnp.float32), W2, w3p])
    b3p = jnp.zeros((4,), jnp.float32).at[:2].set(b3)
    bstack = jnp.stack([b1, b2, b3p])
    flat = jnp.concatenate([wstack.reshape(48), bstack.reshape(12),
                            Wc.reshape(2), bc])
    cfg = jnp.broadcast_to(flat[:, None], (63, 16))

    out_flat, hout, _ = _sc_kernel(z1t, src, dst, cfg)
    out = out_flat[:N_NODES][:, None]
    h = hout[:, :N_NODES].T
    return out, h


# speedup vs baseline: 61.6791x; 1.0484x over previous
import functools

import jax
import jax.numpy as jnp
from jax import lax
from jax.experimental import pallas as pl
from jax.experimental.pallas import tpu as pltpu
from jax.experimental.pallas import tpu_sc as plsc

N_NODES = 10000
N_EDGES = 320000
D_FEAT = 128
NPAD = 10240
NTILES = 16
SLICE = NPAD // NTILES
EPT = N_EDGES // NTILES
EPASS = EPT // 2
NGROUPS = EPASS // 16
NCHUNK = SLICE // 16


def _matmul_tc(w1t_pad, x):
    def body(w_ref, x_ref, o_ref):
        o_ref[...] = jax.lax.dot_general(
            w_ref[...], x_ref[...], (((1,), (1,)), ((), ())),
            preferred_element_type=jnp.float32)

    return pl.pallas_call(
        body,
        grid=(NPAD // 1024,),
        in_specs=[
            pl.BlockSpec((8, D_FEAT), lambda i: (0, 0)),
            pl.BlockSpec((1024, D_FEAT), lambda i: (i, 0)),
        ],
        out_specs=pl.BlockSpec((8, 1024), lambda i: (0, i)),
        out_shape=jax.ShapeDtypeStruct((8, NPAD), jnp.float32),
    )(w1t_pad, x)


def _tanh(v):
    return 2.0 / (1.0 + jnp.exp(-2.0 * v)) - 1.0


def _rsqrt(d):
    i = plsc.bitcast(d, jnp.int32)
    i = jnp.int32(0x5F3759DF) - (i >> 1)
    y = plsc.bitcast(i, jnp.float32)
    for _ in range(3):
        y = y * (1.5 - 0.5 * d * y * y)
    return y


def _hist_body(dsth, dinv_o, prth_o,
               hist, dstb2, t0, t1, t2, hsem, hsem2):
    s = lax.axis_index("s")
    base_n = s * SLICE
    base_e = s * EPT

    @plsc.parallel_loop(0, NPAD // 16, unroll=8)
    def zh(i):
        hist[pl.ds(i * 16, 16)] = jnp.zeros((16,), jnp.float32)

    for p in range(2):
        pltpu.sync_copy(dsth.at[pl.ds(base_e + p * EPASS, EPASS)], dstb2)

        @plsc.parallel_loop(0, NGROUPS, unroll=8)
        def hg(g):
            dv = dstb2[pl.ds(g * 16, 16)]
            cnt, last = plsc.scan_count(dv)
            plsc.addupdate_scatter(hist, [dv], cnt.astype(jnp.float32),
                                   mask=last)

    pltpu.sync_copy(hist, prth_o.at[s])
    plsc.subcore_barrier()
    pltpu.sync_copy(prth_o.at[0, pl.ds(base_n, SLICE)], t0)
    bufs = [t1, t2]
    sems = [hsem, hsem2]
    cps = [None, None]
    cps[1] = pltpu.async_copy(
        prth_o.at[1, pl.ds(base_n, SLICE)], bufs[1], sems[1])
    for t in range(1, NTILES):
        cur = bufs[t % 2]
        cps[t % 2].wait()
        if t + 1 < NTILES:
            cps[(t + 1) % 2] = pltpu.async_copy(
                prth_o.at[t + 1, pl.ds(base_n, SLICE)],
                bufs[(t + 1) % 2], sems[(t + 1) % 2])

        @plsc.parallel_loop(0, NCHUNK, unroll=4)
        def acc_t(i):
            t0[pl.ds(i * 16, 16)] = (t0[pl.ds(i * 16, 16)]
                                     + cur[pl.ds(i * 16, 16)])

    @plsc.parallel_loop(0, NCHUNK, unroll=4)
    def dv_(i):
        t0[pl.ds(i * 16, 16)] = _rsqrt(t0[pl.ds(i * 16, 16)] + 1.0)

    pltpu.sync_copy(t0, dinv_o.at[pl.ds(base_n, SLICE)])


_hist_kernel = functools.partial(
    pl.kernel,
    out_type=[
        jax.ShapeDtypeStruct((NPAD,), jnp.float32),
        jax.ShapeDtypeStruct((NTILES, NPAD), jnp.float32),
    ],
    mesh=plsc.VectorSubcoreMesh(
        core_axis_name="c", subcore_axis_name="s", num_cores=1),
    compiler_params=pltpu.CompilerParams(needs_layout_passes=False),
    scratch_types=[
        pltpu.VMEM((NPAD,), jnp.float32),
        pltpu.VMEM((EPASS,), jnp.int32),
        pltpu.VMEM((SLICE,), jnp.float32),
        pltpu.VMEM((SLICE,), jnp.float32),
        pltpu.VMEM((SLICE,), jnp.float32),
        pltpu.SemaphoreType.DMA,
        pltpu.SemaphoreType.DMA,
    ],
)(_hist_body)


def _sc_body(z1t, srch, dsth, cfg, dinvh,
             out_o, hout_o, prt_o,
             A, B, dinvb, srcb, dstb, tmp, tmp2, tmp3,
             cfgv, sh1, dma_sem, dma_sem2):
    s = lax.axis_index("s")
    base_n = s * SLICE
    base_e = s * EPT

    pltpu.sync_copy(cfg, cfgv)

    def _zero_B():
        @plsc.parallel_loop(0, NPAD // 16, unroll=8)
        def zb(i):
            for f in range(4):
                B[f, pl.ds(i * 16, 16)] = jnp.zeros((16,), jnp.float32)

    def _reduce_B():
        with jax.named_scope("reduceB"):
            pltpu.sync_copy(B, prt_o.at[s])
            plsc.subcore_barrier()
            pltpu.sync_copy(prt_o.at[0, :, pl.ds(base_n, SLICE)], tmp)
            bufs = [tmp2, tmp3]
            sems = [dma_sem, dma_sem2]
            cps = [None, None]
            cps[1] = pltpu.async_copy(
                prt_o.at[1, :, pl.ds(base_n, SLICE)], bufs[1], sems[1])
            for t in range(1, NTILES):
                cur = bufs[t % 2]
                cps[t % 2].wait()
                if t + 1 < NTILES:
                    cps[(t + 1) % 2] = pltpu.async_copy(
                        prt_o.at[t + 1, :, pl.ds(base_n, SLICE)],
                        bufs[(t + 1) % 2], sems[(t + 1) % 2])

                @plsc.parallel_loop(0, NCHUNK, unroll=4)
                def acc_t(i):
                    for f in range(4):
                        tmp[f, pl.ds(i * 16, 16)] = (
                            tmp[f, pl.ds(i * 16, 16)]
                            + cur[f, pl.ds(i * 16, 16)])

    pltpu.sync_copy(dinvh, dinvb)

    with jax.named_scope("loadz1"):
        for f in range(4):
            pltpu.sync_copy(z1t.at[f], A.at[f])

    def layer_body(l, _):
        with jax.named_scope("mm"):
            @plsc.parallel_loop(0, NCHUNK, unroll=4)
            def mm(i):
                o = base_n + i * 16
                for j in range(4):
                    acc = jnp.zeros((16,), jnp.float32)
                    for ii in range(4):
                        acc = (acc
                               + cfgv[l * 16 + ii * 4 + j] * A[ii, pl.ds(o, 16)])
                    tmp[j, pl.ds(i * 16, 16)] = acc

            for j in range(4):
                pltpu.sync_copy(tmp.at[j], sh1.at[j, pl.ds(base_n, SLICE)])
            plsc.subcore_barrier()
            for j in range(4):
                pltpu.sync_copy(sh1.at[j], A.at[j])

        with jax.named_scope("zeroB"):
            _zero_B()

        for p in range(2):
            with jax.named_scope("edgeload"):
                pltpu.sync_copy(srch.at[pl.ds(base_e + p * EPASS, EPASS)],
                                srcb)
                pltpu.sync_copy(dsth.at[pl.ds(base_e + p * EPASS, EPASS)],
                                dstb)

            with jax.named_scope("edges"):
                @plsc.parallel_loop(0, NGROUPS, unroll=8,
                                    carry=jnp.zeros((16,), jnp.int32))
                def eg(g, cmax):
                    o = g * 16
                    sv = srcb[pl.ds(o, 16)]
                    dv = dstb[pl.ds(o, 16)]
                    nrm = (plsc.load_gather(dinvb, [sv])
                           * plsc.load_gather(dinvb, [dv]))
                    cnt, _ = plsc.scan_count(dv)
                    m1 = cnt == 1
                    m2 = cnt == 2
                    for f in range(4):
                        fs = jnp.full((16,), f, jnp.int32)
                        msg = plsc.load_gather(A, [fs, sv]) * nrm
                        plsc.addupdate_scatter(B, [fs, dv], msg, mask=m1)
                        plsc.addupdate_scatter(B, [fs, dv], msg, mask=m2)
                    return jnp.maximum(cmax, cnt)

                ovf = jnp.max(eg)

                @pl.when(ovf > 2)
                def _fixup():
                    def fg(g, _):
                        o = g * 16
                        sv = srcb[pl.ds(o, 16)]
                        dv = dstb[pl.ds(o, 16)]
                        nrm = (plsc.load_gather(dinvb, [sv])
                               * plsc.load_gather(dinvb, [dv]))
                        cnt, _ = plsc.scan_count(dv)
                        maxc = jnp.max(cnt)

                        def dup(j, __):
                            m = cnt == j
                            for f in range(4):
                                fs = jnp.full((16,), f, jnp.int32)
                                msg = plsc.load_gather(A, [fs, sv]) * nrm
                                plsc.addupdate_scatter(B, [fs, dv], msg,
                                                       mask=m)
                            return 0

                        lax.fori_loop(3, maxc + 1, dup, 0)
                        return 0

                    lax.fori_loop(0, NGROUPS, fg, 0)

        _reduce_B()

        with jax.named_scope("post"):
            @plsc.parallel_loop(0, NCHUNK, unroll=4)
            def post(i):
                o16 = i * 16
                on = base_n + o16
                di = dinvb[pl.ds(on, 16)]
                d2 = di * di
                for f in range(4):
                    a = tmp[f, pl.ds(o16, 16)]
                    z = A[f, pl.ds(on, 16)]
                    tmp[f, pl.ds(o16, 16)] = _tanh(a + d2 * z + cfgv[48 + l * 4 + f])

            for f in range(4):
                pltpu.sync_copy(tmp.at[f], sh1.at[f, pl.ds(base_n, SLICE)])
            plsc.subcore_barrier()
            for f in range(4):
                pltpu.sync_copy(sh1.at[f], A.at[f])
            plsc.subcore_barrier()
        return 0

    lax.fori_loop(0, 3, layer_body, 0)

    @plsc.parallel_loop(0, NCHUNK, unroll=4)
    def cls(i):
        o16 = i * 16
        on = base_n + o16
        h0 = A[0, pl.ds(on, 16)]
        h1 = A[1, pl.ds(on, 16)]
        tmp[0, pl.ds(o16, 16)] = cfgv[60] * h0 + cfgv[61] * h1 + cfgv[62]

    pltpu.sync_copy(tmp.at[0], out_o.at[pl.ds(base_n, SLICE)])
    for f in range(2):
        pltpu.sync_copy(A.at[f, pl.ds(base_n, SLICE)],
                        hout_o.at[f, pl.ds(base_n, SLICE)])


_sc_kernel = functools.partial(
    pl.kernel,
    out_type=[
        jax.ShapeDtypeStruct((NPAD,), jnp.float32),
        jax.ShapeDtypeStruct((2, NPAD), jnp.float32),
        jax.ShapeDtypeStruct((NTILES, 4, NPAD), jnp.float32),
    ],
    mesh=plsc.VectorSubcoreMesh(
        core_axis_name="c", subcore_axis_name="s", num_cores=1),
    compiler_params=pltpu.CompilerParams(needs_layout_passes=False),
    scratch_types=[
        pltpu.VMEM((4, NPAD), jnp.float32),
        pltpu.VMEM((4, NPAD), jnp.float32),
        pltpu.VMEM((NPAD,), jnp.float32),
        pltpu.VMEM((EPASS,), jnp.int32),
        pltpu.VMEM((EPASS,), jnp.int32),
        pltpu.VMEM((4, SLICE), jnp.float32),
        pltpu.VMEM((4, SLICE), jnp.float32),
        pltpu.VMEM((4, SLICE), jnp.float32),
        pltpu.VMEM((63, 16), jnp.float32),
        pltpu.VMEM_SHARED((4, NPAD), jnp.float32),
        pltpu.SemaphoreType.DMA,
        pltpu.SemaphoreType.DMA,
    ],
)(_sc_body)


def kernel(x, edge_index, W1, b1, W2, b2, W3, b3, Wc, bc):
    w1t = jnp.zeros((8, D_FEAT), jnp.float32).at[:4].set(W1.T)
    z1t = _matmul_tc(w1t, x)

    src = edge_index[0].astype(jnp.int32)
    dst = edge_index[1].astype(jnp.int32)

    w3p = jnp.zeros((4, 4), jnp.float32).at[:, :2].set(W3)
    wstack = jnp.stack([jnp.eye(4, dtype=jnp.float32), W2, w3p])
    b3p = jnp.zeros((4,), jnp.float32).at[:2].set(b3)
    bstack = jnp.stack([b1, b2, b3p])
    flat = jnp.concatenate([wstack.reshape(48), bstack.reshape(12),
                            Wc.reshape(2), bc])
    cfg = jnp.broadcast_to(flat[:, None], (63, 16))

    dinv, _ = _hist_kernel(dst)
    out_flat, hout, _ = _sc_kernel(z1t, src, dst, cfg, dinv)
    out = out_flat[:N_NODES][:, None]
    h = hout[:, :N_NODES].T
    return out, h
